# baseline (speedup 1.0000x reference)
import jax
import jax.numpy as jnp
from jax.experimental import pallas as pl
from jax.experimental.pallas import tpu as pltpu

B = 8
N = 64
K = 16
T = 128
D = 768
NBUF = 32
LAG = 16


def _fused_body(ro_ref, mask_ref, reps_ref, selmask_ref, out_ref,
                idx_vmem, idx_smem, buf, sem_idx, in_sems, out_sems):
    scores = ro_ref[:, :, 1]
    s_i = scores[:, :, None]
    s_j = scores[:, None, :]
    j_iota = jax.lax.broadcasted_iota(jnp.int32, (B, N, N), 2)
    i_iota = jax.lax.broadcasted_iota(jnp.int32, (B, N, N), 1)
    beats = (s_j > s_i) | ((s_j == s_i) & (j_iota < i_iota))
    rank = beats.astype(jnp.int32).sum(axis=2)
    k_iota = jax.lax.broadcasted_iota(jnp.int32, (B, K, N), 1)
    eq = rank[:, None, :] == k_iota
    n_iota = jax.lax.broadcasted_iota(jnp.int32, (B, K, N), 2)
    idx_vmem[:, :] = jnp.where(eq, n_iota, 0).sum(axis=2)
    pltpu.async_copy(idx_vmem, idx_smem, sem_idx).wait()

    G = B * K * 2
    in_h = [None] * G
    out_h = [None] * G

    H = T // 2

    def start_in(g):
        bk, h = divmod(g, 2)
        b, k = divmod(bk, K)
        i = idx_smem[b, k]
        in_h[g] = pltpu.async_copy(
            reps_ref.at[b, i, pl.ds(h * H, H)],
            buf.at[g % NBUF],
            in_sems.at[g % NBUF],
        )

    def start_out(g):
        bk, h = divmod(g, 2)
        b, k = divmod(bk, K)
        in_h[g].wait()
        out_h[g] = pltpu.async_copy(
            buf.at[g % NBUF],
            out_ref.at[b, k, pl.ds(h * H, H)],
            out_sems.at[g % NBUF],
        )

    for g in range(G):
        if g >= NBUF:
            out_h[g - NBUF].wait()
        start_in(g)
        if g >= LAG:
            start_out(g - LAG)

    eqf = eq.astype(jnp.float32)
    for b in range(B):
        selmask_ref[b] = jnp.dot(
            eqf[b], mask_ref[b], preferred_element_type=jnp.float32
        )

    for g in range(G - LAG, G):
        start_out(g)
    for g in range(G - NBUF, G):
        out_h[g].wait()


def kernel(token_reps, token_mask, rationale_out):
    sel_mask, sel_reps = pl.pallas_call(
        _fused_body,
        in_specs=[
            pl.BlockSpec(memory_space=pltpu.MemorySpace.VMEM),
            pl.BlockSpec(memory_space=pltpu.MemorySpace.VMEM),
            pl.BlockSpec(memory_space=pltpu.MemorySpace.HBM),
        ],
        out_specs=[
            pl.BlockSpec(memory_space=pltpu.MemorySpace.VMEM),
            pl.BlockSpec(memory_space=pltpu.MemorySpace.HBM),
        ],
        out_shape=(
            jax.ShapeDtypeStruct((B, K, T), jnp.float32),
            jax.ShapeDtypeStruct((B, K, T, D), jnp.float32),
        ),
        scratch_shapes=[
            pltpu.VMEM((B, K), jnp.int32),
            pltpu.SMEM((B, K), jnp.int32),
            pltpu.VMEM((NBUF, T // 2, D), jnp.float32),
            pltpu.SemaphoreType.DMA,
            pltpu.SemaphoreType.DMA((NBUF,)),
            pltpu.SemaphoreType.DMA((NBUF,)),
        ],
    )(rationale_out, token_mask, token_reps)
    return (sel_reps, sel_mask)


# speedup vs baseline: 1.0304x; 1.0304x over previous
import jax
import jax.numpy as jnp
from jax.experimental import pallas as pl
from jax.experimental.pallas import tpu as pltpu

B = 8
N = 64
K = 16
T = 128
D = 768
NBUF = 32
GRP = 8
GLAG = 2
NGRP = (B * K) // GRP
GSEM = NBUF // GRP


def _fused_body(ro_ref, mask_ref, reps_ref, selmask_ref, out_ref,
                idx_vmem, idx_smem, buf, sem_idx, in_sems, out_sems):
    scores = ro_ref[:, :, 1]
    s_i = scores[:, :, None]
    s_j = scores[:, None, :]
    j_iota = jax.lax.broadcasted_iota(jnp.int32, (B, N, N), 2)
    i_iota = jax.lax.broadcasted_iota(jnp.int32, (B, N, N), 1)
    beats = (s_j > s_i) | ((s_j == s_i) & (j_iota < i_iota))
    rank = beats.astype(jnp.int32).sum(axis=2)
    k_iota = jax.lax.broadcasted_iota(jnp.int32, (B, K, N), 1)
    eq = rank[:, None, :] == k_iota
    n_iota = jax.lax.broadcasted_iota(jnp.int32, (B, K, N), 2)
    idx_vmem[:, :] = jnp.where(eq, n_iota, 0).sum(axis=2)
    pltpu.async_copy(idx_vmem, idx_smem, sem_idx).wait()

    G = B * K
    in_h = [None] * G
    out_h = [None] * NGRP

    def start_in(g):
        b, k = divmod(g, K)
        i = idx_smem[b, k]
        in_h[g] = pltpu.async_copy(
            reps_ref.at[b, i], buf.at[g % NBUF], in_sems.at[g % NBUF]
        )

    def start_out(q):
        g0 = q * GRP
        for t in range(GRP):
            in_h[g0 + t].wait()
        b, k0 = divmod(g0, K)
        out_h[q] = pltpu.async_copy(
            buf.at[pl.ds((g0 % NBUF), GRP)],
            out_ref.at[b, pl.ds(k0, GRP)],
            out_sems.at[q % GSEM],
        )

    for q in range(NGRP):
        for t in range(GRP):
            g = q * GRP + t
            if g >= NBUF and g % GRP == 0:
                out_h[(g - NBUF) // GRP].wait()
            start_in(g)
        if q >= GLAG:
            start_out(q - GLAG)

    eqf = eq.astype(jnp.float32)
    for b in range(B):
        selmask_ref[b] = jnp.dot(
            eqf[b], mask_ref[b], preferred_element_type=jnp.float32
        )

    for q in range(NGRP - GLAG, NGRP):
        start_out(q)
    for q in range(NGRP - GSEM, NGRP):
        out_h[q].wait()


def kernel(token_reps, token_mask, rationale_out):
    sel_mask, sel_reps = pl.pallas_call(
        _fused_body,
        in_specs=[
            pl.BlockSpec(memory_space=pltpu.MemorySpace.VMEM),
            pl.BlockSpec(memory_space=pltpu.MemorySpace.VMEM),
            pl.BlockSpec(memory_space=pltpu.MemorySpace.HBM),
        ],
        out_specs=[
            pl.BlockSpec(memory_space=pltpu.MemorySpace.VMEM),
            pl.BlockSpec(memory_space=pltpu.MemorySpace.HBM),
        ],
        out_shape=(
            jax.ShapeDtypeStruct((B, K, T), jnp.float32),
            jax.ShapeDtypeStruct((B, K, T, D), jnp.float32),
        ),
        scratch_shapes=[
            pltpu.VMEM((B, K), jnp.int32),
            pltpu.SMEM((B, K), jnp.int32),
            pltpu.VMEM((NBUF, T, D), jnp.float32),
            pltpu.SemaphoreType.DMA,
            pltpu.SemaphoreType.DMA((NBUF,)),
            pltpu.SemaphoreType.DMA((GSEM,)),
        ],
    )(rationale_out, token_mask, token_reps)
    return (sel_reps, sel_mask)
